# baseline (speedup 1.0000x reference)
import functools

import jax
import jax.numpy as jnp
from jax.experimental import pallas as pl
from jax.experimental.pallas import tpu as pltpu

N = 1024
D = 128
BK = 2048


FW = 4
FL = BK // FW


def _nn_kernel(emb_ref, w_ref, b_ref, tab_ref, out_ref, best_ref, blk_ref,
               *, nsteps, blk):
    j = pl.program_id(0)

    tb = tab_ref[...]
    ones_row = jnp.ones((1, D), jnp.float32)
    contract = (((1,), (1,)), ((), ()))
    t2h = 0.5 * jax.lax.dot_general(ones_row, tb * tb, contract,
                                    precision=jax.lax.Precision.HIGHEST,
                                    preferred_element_type=jnp.float32)

    y = (emb_ref[...] - b_ref[...]) / (w_ref[...] + 1e-6)

    mm = jax.lax.dot_general(y, tb, contract,
                             preferred_element_type=jnp.float32)
    s = t2h - mm

    s0, s1 = s[:, :FL], s[:, FL:2 * FL]
    s2, s3 = s[:, 2 * FL:3 * FL], s[:, 3 * FL:]
    c1 = s1 < s0
    c2 = s3 < s2
    m01 = jnp.minimum(s0, s1)
    m23 = jnp.minimum(s2, s3)
    c3 = m23 < m01
    m4 = jnp.minimum(m01, m23)
    k01 = jnp.where(c1, jnp.int32(1), jnp.int32(0))
    k23 = jnp.where(c2, jnp.int32(3), jnp.int32(2))
    k = jnp.where(c3, k23, k01)

    prev = jnp.where(j == 0, jnp.float32(jnp.inf), best_ref[...])
    upd = m4 < prev
    best_ref[...] = jnp.minimum(m4, prev)
    blk_ref[...] = jnp.where(upd, j * FW + k, blk_ref[...])

    @pl.when(j == nsteps - 1)
    def _done():
        m = best_ref[...]
        rowmin = jnp.min(m, axis=1, keepdims=True)
        lane = jax.lax.broadcasted_iota(jnp.int32, (1, FL), 1)
        gcol = blk_ref[...] * FL + lane
        big = jnp.int32(2147483647)
        cand = jnp.where(m == rowmin, gcol, big)
        out_ref[...] = jnp.min(cand, axis=1, keepdims=True)


@jax.jit
def kernel(embeddings, ln_weight, ln_bias, table):
    vocab = table.shape[0]
    nsteps = pl.cdiv(vocab, BK)
    padded = nsteps * BK
    if padded != vocab:
        table = jnp.pad(table, ((0, padded - vocab), (0, 0)), mode="edge")

    out = pl.pallas_call(
        functools.partial(_nn_kernel, nsteps=nsteps, blk=BK),
        grid=(nsteps,),
        in_specs=[
            pl.BlockSpec((N, D), lambda j: (0, 0)),
            pl.BlockSpec((1, D), lambda j: (0, 0)),
            pl.BlockSpec((1, D), lambda j: (0, 0)),
            pl.BlockSpec((BK, D), lambda j: (j, 0)),
        ],
        out_specs=pl.BlockSpec((N, 1), lambda j: (0, 0)),
        out_shape=jax.ShapeDtypeStruct((N, 1), jnp.int32),
        scratch_shapes=[
            pltpu.VMEM((N, FL), jnp.float32),
            pltpu.VMEM((N, FL), jnp.int32),
        ],
    )(embeddings, ln_weight[None, :], ln_bias[None, :], table)
    return out[:, 0]
